# baseline (speedup 1.0000x reference)
import functools

import numpy as np
import jax
import jax.numpy as jnp
from jax import lax
from jax.experimental import pallas as pl
from jax.experimental.pallas import tpu as pltpu
from jax.experimental.pallas import tpu_sc as plsc

N_BATCH = 16384
N_PARTICLES = 256
N_DIM = 3
ROW = N_PARTICLES * N_DIM
CG_ROW = (N_PARTICLES // 4) * N_DIM
NCG_ROW = ROW - CG_ROW

NW = 32
ROWS_PER_W = N_BATCH // NW
CHUNK = 32
NCHUNK = ROWS_PER_W // CHUNK

_j = np.arange(48)
_cg_pat = (_j // 3) * 12 + _j % 3
_t = np.arange(144)
_ncg_pat = (_t // 9) * 12 + 3 + _t % 9
_PAT = np.concatenate([_cg_pat, _ncg_pat]).astype(np.int32)


def _sc_split(x_flat, pat):
    mesh = plsc.VectorSubcoreMesh(core_axis_name="c", subcore_axis_name="s")

    @functools.partial(
        pl.kernel,
        mesh=mesh,
        compiler_params=pltpu.CompilerParams(needs_layout_passes=False),
        out_type=(
            jax.ShapeDtypeStruct((N_BATCH * CG_ROW,), jnp.float32),
            jax.ShapeDtypeStruct((N_BATCH * NCG_ROW,), jnp.float32),
        ),
        scratch_types=[
            pltpu.VMEM((192,), jnp.int32),
            pltpu.VMEM((CHUNK * ROW,), jnp.float32),
            pltpu.VMEM((CHUNK * CG_ROW,), jnp.float32),
            pltpu.VMEM((CHUNK * NCG_ROW,), jnp.float32),
        ],
    )
    def k(x_hbm, pat_hbm, cg_hbm, ncg_hbm, pat_v, vin, vcg, vncg):
        wid = lax.axis_index("s") * 2 + lax.axis_index("c")
        base_row = wid * ROWS_PER_W

        pltpu.sync_copy(pat_hbm, pat_v)
        cg_pats = [pat_v[pl.ds(v * 16, 16)] for v in range(3)]
        ncg_pats = [pat_v[pl.ds(48 + v * 16, 16)] for v in range(9)]

        def row_body(r, _):
            src0 = r * ROW
            for m in range(4):
                off = src0 + m * 192
                for v in range(3):
                    val = plsc.load_gather(vin, [cg_pats[v] + off])
                    vcg[pl.ds(r * CG_ROW + m * 48 + v * 16, 16)] = val
                for v in range(9):
                    val = plsc.load_gather(vin, [ncg_pats[v] + off])
                    vncg[pl.ds(r * NCG_ROW + m * 144 + v * 16, 16)] = val
            return _

        def chunk_body(c, _):
            row0 = base_row + c * CHUNK
            pltpu.sync_copy(x_hbm.at[pl.ds(row0 * ROW, CHUNK * ROW)], vin)
            lax.fori_loop(0, CHUNK, row_body, None)
            pltpu.sync_copy(vcg, cg_hbm.at[pl.ds(row0 * CG_ROW, CHUNK * CG_ROW)])
            pltpu.sync_copy(vncg,
                            ncg_hbm.at[pl.ds(row0 * NCG_ROW, CHUNK * NCG_ROW)])
            return _

        lax.fori_loop(0, NCHUNK, chunk_body, None)

    return k(x_flat, pat)


@jax.jit
def kernel(input_coords):
    x_flat = input_coords.reshape(N_BATCH * ROW)
    cg_flat, ncg_flat = _sc_split(x_flat, jnp.asarray(_PAT))
    cg = cg_flat.reshape(N_BATCH, N_PARTICLES // 4, N_DIM)
    ncg = ncg_flat.reshape(N_BATCH, N_PARTICLES - N_PARTICLES // 4, N_DIM)
    return (cg, ncg)


# speedup vs baseline: 22.9010x; 22.9010x over previous
import functools

import numpy as np
import jax
import jax.numpy as jnp
from jax import lax
from jax.experimental import pallas as pl
from jax.experimental.pallas import tpu as pltpu
from jax.experimental.pallas import tpu_sc as plsc

N_BATCH = 16384
N_PARTICLES = 256
N_DIM = 3
N_CG = N_PARTICLES // 4
N_NCG = N_PARTICLES - N_CG

NW = 32
BLOCKS_PER_W = 4

IN_PLANE = N_BATCH * N_PARTICLES
SLAB = 16 * 2 * 8 * 128
CG_TILEROW = 128 * 8 * 128
NCG_TILEROW = 128 * 8 * 128


def _sc_split(x_flat):
    mesh = plsc.VectorSubcoreMesh(core_axis_name="c", subcore_axis_name="s")

    @functools.partial(
        pl.kernel,
        mesh=mesh,
        compiler_params=pltpu.CompilerParams(needs_layout_passes=False),
        out_type=(
            jax.ShapeDtypeStruct((N_DIM * N_CG * N_BATCH,), jnp.float32),
            jax.ShapeDtypeStruct((N_DIM * N_NCG * N_BATCH,), jnp.float32),
        ),
        scratch_types=[
            pltpu.VMEM((SLAB,), jnp.float32),
            pltpu.VMEM((64 * 128,), jnp.float32),
            pltpu.VMEM((192 * 128,), jnp.float32),
            pltpu.SemaphoreType.DMA,
        ],
    )
    def k(x_hbm, cg_hbm, ncg_hbm, vin, vcg, vncg, sem):
        wid = lax.axis_index("s") * 2 + lax.axis_index("c")

        i = lax.iota(jnp.int32, 16)
        pat0 = (i // 8) * 2048 + (i % 8) * 128
        pats = [pat0 + v * 4096 for v in range(8)]

        def block_body(c, _):
            tbo = wid * BLOCKS_PER_W + c
            for p in range(N_DIM):
                src0 = p * IN_PLANE + tbo * SLAB
                pltpu.sync_copy(x_hbm.at[pl.ds(src0, SLAB)], vin)

                def row_body(g, _):
                    base = g * 4 + (g // 32) * 896
                    for t in range(4):
                        qoff = base + t
                        if t == 0:
                            dst0 = g * 128
                            dref = vcg
                        else:
                            dst0 = (3 * g + t - 1) * 128
                            dref = vncg
                        for v in range(8):
                            val = plsc.load_gather(vin, [pats[v] + qoff])
                            dref[pl.ds(dst0 + 16 * v, 16)] = val
                    return _

                lax.fori_loop(0, 64, row_body, None)

                descs = []
                for tk in range(8):
                    d0 = ((p * 8 + tk) * 128 + tbo) * 1024
                    descs.append(pltpu.async_copy(
                        vcg.at[pl.ds(tk * 1024, 1024)],
                        cg_hbm.at[pl.ds(d0, 1024)], sem))
                for tj in range(24):
                    d0 = ((p * 24 + tj) * 128 + tbo) * 1024
                    descs.append(pltpu.async_copy(
                        vncg.at[pl.ds(tj * 1024, 1024)],
                        ncg_hbm.at[pl.ds(d0, 1024)], sem))
                for d in descs:
                    d.wait()
            return _

        lax.fori_loop(0, BLOCKS_PER_W, block_body, None)

    return k(x_flat)


@jax.jit
def kernel(input_coords):
    x5 = (input_coords.transpose(2, 0, 1)
          .reshape(N_DIM, 2048, 8, 2, 128)
          .transpose(0, 1, 3, 2, 4))
    x_flat = x5.reshape(N_DIM * IN_PLANE)
    cg_flat, ncg_flat = _sc_split(x_flat)
    cg = (cg_flat.reshape(N_DIM, 8, 128, 8, 128)
          .transpose(0, 1, 3, 2, 4)
          .reshape(N_DIM, N_CG, N_BATCH)
          .transpose(2, 1, 0))
    ncg = (ncg_flat.reshape(N_DIM, 24, 128, 8, 128)
           .transpose(0, 1, 3, 2, 4)
           .reshape(N_DIM, N_NCG, N_BATCH)
           .transpose(2, 1, 0))
    return (cg, ncg)


# speedup vs baseline: 25.1742x; 1.0993x over previous
import functools

import numpy as np
import jax
import jax.numpy as jnp
from jax import lax
from jax.experimental import pallas as pl
from jax.experimental.pallas import tpu as pltpu
from jax.experimental.pallas import tpu_sc as plsc

N_BATCH = 16384
N_PARTICLES = 256
N_DIM = 3
N_CG = N_PARTICLES // 4
N_NCG = N_PARTICLES - N_CG

NW = 32
BLOCKS_PER_W = 4
NSLAB = BLOCKS_PER_W * N_DIM

IN_PLANE = N_BATCH * N_PARTICLES
SLAB = 16 * 2 * 8 * 128


def _sc_split(x_flat):
    mesh = plsc.VectorSubcoreMesh(core_axis_name="c", subcore_axis_name="s")

    @functools.partial(
        pl.kernel,
        mesh=mesh,
        compiler_params=pltpu.CompilerParams(needs_layout_passes=False),
        out_type=(
            jax.ShapeDtypeStruct((N_DIM * 8, 128, 8, 128), jnp.float32),
            jax.ShapeDtypeStruct((N_DIM * 24, 128, 8, 128), jnp.float32),
        ),
        scratch_types=[
            pltpu.VMEM((SLAB,), jnp.float32),
            pltpu.VMEM((SLAB,), jnp.float32),
            pltpu.VMEM((8, 8, 128), jnp.float32),
            pltpu.VMEM((8, 8, 128), jnp.float32),
            pltpu.VMEM((24, 8, 128), jnp.float32),
            pltpu.VMEM((24, 8, 128), jnp.float32),
            pltpu.SemaphoreType.DMA,
            pltpu.SemaphoreType.DMA,
            pltpu.SemaphoreType.DMA,
            pltpu.SemaphoreType.DMA,
        ],
    )
    def k(x_hbm, cg_hbm, ncg_hbm, vin0, vin1, vcg0, vcg1, vncg0, vncg1,
          sin0, sin1, sout0, sout1):
        wid = lax.axis_index("s") * 2 + lax.axis_index("c")
        vins = [vin0, vin1]
        vcgs = [vcg0, vcg1]
        vncgs = [vncg0, vncg1]
        sins = [sin0, sin1]
        souts = [sout0, sout1]

        i = lax.iota(jnp.int32, 16)
        pat0 = (i // 8) * 2048 + (i % 8) * 128
        pats = [pat0 + v * 4096 for v in range(8)]

        def slab_coords(s):
            c, p = divmod(s, 3)
            tbo = wid * BLOCKS_PER_W + c
            return p, tbo

        def start_in(s):
            p, tbo = slab_coords(s)
            src0 = p * IN_PLANE + tbo * SLAB
            return pltpu.async_copy(
                x_hbm.at[pl.ds(src0, SLAB)], vins[s % 2], sins[s % 2])

        in_descs = {0: start_in(0)}
        out_descs = {0: [], 1: []}
        for s in range(NSLAB):
            b = s % 2
            if s + 1 < NSLAB:
                in_descs[s + 1] = start_in(s + 1)
            in_descs[s].wait()
            for d in out_descs[b]:
                d.wait()
            vin, vcg, vncg = vins[b], vcgs[b], vncgs[b]

            def row_body(gi, _, vin=vin, vcg=vcg, vncg=vncg):
                for u in range(2):
                    g = gi * 2 + u
                    base = g * 4 + (g // 32) * 896
                    for t in range(4):
                        qoff = base + t
                        if t == 0:
                            row, sub = g // 8, g % 8
                            dref = vcg
                        else:
                            j = 3 * g + t - 1
                            row, sub = j // 8, j % 8
                            dref = vncg
                        for v in range(8):
                            val = plsc.load_gather(vin, [pats[v] + qoff])
                            dref[row, sub, pl.ds(16 * v, 16)] = val
                return _

            lax.fori_loop(0, 32, row_body, None)

            p, tbo = slab_coords(s)
            out_descs[b] = [
                pltpu.async_copy(vcg, cg_hbm.at[pl.ds(p * 8, 8), tbo],
                                 souts[b]),
                pltpu.async_copy(vncg, ncg_hbm.at[pl.ds(p * 24, 24), tbo],
                                 souts[b]),
            ]
        for b in range(2):
            for d in out_descs[b]:
                d.wait()

    return k(x_flat)


@jax.jit
def kernel(input_coords):
    x5 = (input_coords.transpose(2, 0, 1)
          .reshape(N_DIM, 2048, 8, 2, 128)
          .transpose(0, 1, 3, 2, 4))
    x_flat = x5.reshape(N_DIM * IN_PLANE)
    cg_t, ncg_t = _sc_split(x_flat)
    cg = (cg_t.reshape(N_DIM, 8, 128, 8, 128)
          .transpose(0, 1, 3, 2, 4)
          .reshape(N_DIM, N_CG, N_BATCH)
          .transpose(2, 1, 0))
    ncg = (ncg_t.reshape(N_DIM, 24, 128, 8, 128)
           .transpose(0, 1, 3, 2, 4)
           .reshape(N_DIM, N_NCG, N_BATCH)
           .transpose(2, 1, 0))
    return (cg, ncg)


# speedup vs baseline: 39.2043x; 1.5573x over previous
import functools

import numpy as np
import jax
import jax.numpy as jnp
from jax import lax
from jax.experimental import pallas as pl
from jax.experimental.pallas import tpu as pltpu
from jax.experimental.pallas import tpu_sc as plsc

N_BATCH = 16384
N_PARTICLES = 256
N_DIM = 3
N_CG = N_PARTICLES // 4
N_NCG = N_PARTICLES - N_CG

NW = 32
BLOCKS_PER_W = 4
NSLAB = BLOCKS_PER_W * N_DIM

IN_PLANE = N_BATCH * N_PARTICLES
SLAB = 16 * 2 * 8 * 128


def _sc_split(x_flat):
    mesh = plsc.VectorSubcoreMesh(core_axis_name="c", subcore_axis_name="s")

    @functools.partial(
        pl.kernel,
        mesh=mesh,
        compiler_params=pltpu.CompilerParams(needs_layout_passes=False),
        out_type=(
            jax.ShapeDtypeStruct((N_DIM * 8, 128, 8, 128), jnp.float32),
            jax.ShapeDtypeStruct((N_DIM * 24, 128, 8, 128), jnp.float32),
        ),
        scratch_types=[
            pltpu.VMEM((SLAB,), jnp.float32),
            pltpu.VMEM((SLAB,), jnp.float32),
            pltpu.VMEM((8, 8, 128), jnp.float32),
            pltpu.VMEM((8, 8, 128), jnp.float32),
            pltpu.VMEM((24, 8, 128), jnp.float32),
            pltpu.SemaphoreType.DMA,
            pltpu.SemaphoreType.DMA,
            pltpu.SemaphoreType.DMA,
            pltpu.SemaphoreType.DMA,
            pltpu.SemaphoreType.DMA,
        ],
    )
    def k(x_hbm, cg_hbm, ncg_hbm, vin0, vin1, vcg0, vcg1, vncg,
          sin0, sin1, sout0, sout1, sncg):
        wid = lax.axis_index("s") * 2 + lax.axis_index("c")
        vins = [vin0, vin1]
        vcgs = [vcg0, vcg1]
        sins = [sin0, sin1]
        souts = [sout0, sout1]

        i = lax.iota(jnp.int32, 16)
        pat0 = (i // 8) * 2048 + (i % 8) * 128
        pats = [pat0 + v * 4096 for v in range(8)]

        def slab_coords(s):
            c, p = divmod(s, 3)
            tbo = wid * BLOCKS_PER_W + c
            return p, tbo

        def start_in(s):
            p, tbo = slab_coords(s)
            src0 = p * IN_PLANE + tbo * SLAB
            return pltpu.async_copy(
                x_hbm.at[pl.ds(src0, SLAB)], vins[s % 2], sins[s % 2])

        in_descs = {0: start_in(0)}
        out_descs = {0: [], 1: []}
        ncg_desc = [None]
        for s in range(NSLAB):
            b = s % 2
            if s + 1 < NSLAB:
                in_descs[s + 1] = start_in(s + 1)
            in_descs[s].wait()
            for d in out_descs[b]:
                d.wait()
            if ncg_desc[0] is not None:
                ncg_desc[0].wait()
            vin, vcg = vins[b], vcgs[b]

            @plsc.parallel_loop(0, 64, unroll=2)
            def row_body(g, vin=vin, vcg=vcg, vncg=vncg):
                base = g * 4 + (g // 32) * 896
                for t in range(4):
                    qoff = base + t
                    if t == 0:
                        row, sub = g // 8, g % 8
                        dref = vcg
                    else:
                        j = 3 * g + t - 1
                        row, sub = j // 8, j % 8
                        dref = vncg
                    for v in range(8):
                        val = plsc.load_gather(vin, [pats[v] + qoff])
                        dref[row, sub, pl.ds(16 * v, 16)] = val

            p, tbo = slab_coords(s)
            out_descs[b] = [
                pltpu.async_copy(vcg, cg_hbm.at[pl.ds(p * 8, 8), tbo],
                                 souts[b]),
            ]
            ncg_desc[0] = pltpu.async_copy(
                vncg, ncg_hbm.at[pl.ds(p * 24, 24), tbo], sncg)
        for b in range(2):
            for d in out_descs[b]:
                d.wait()
        ncg_desc[0].wait()

    return k(x_flat)


@jax.jit
def kernel(input_coords):
    x5 = (input_coords.transpose(2, 0, 1)
          .reshape(N_DIM, 2048, 8, 2, 128)
          .transpose(0, 1, 3, 2, 4))
    x_flat = x5.reshape(N_DIM * IN_PLANE)
    cg_t, ncg_t = _sc_split(x_flat)
    cg = (cg_t.reshape(N_DIM, 8, 128, 8, 128)
          .transpose(0, 1, 3, 2, 4)
          .reshape(N_DIM, N_CG, N_BATCH)
          .transpose(2, 1, 0))
    ncg = (ncg_t.reshape(N_DIM, 24, 128, 8, 128)
           .transpose(0, 1, 3, 2, 4)
           .reshape(N_DIM, N_NCG, N_BATCH)
           .transpose(2, 1, 0))
    return (cg, ncg)


# speedup vs baseline: 39.2629x; 1.0015x over previous
import functools

import numpy as np
import jax
import jax.numpy as jnp
from jax import lax
from jax.experimental import pallas as pl
from jax.experimental.pallas import tpu as pltpu
from jax.experimental.pallas import tpu_sc as plsc

N_BATCH = 16384
N_PARTICLES = 256
N_DIM = 3
N_CG = N_PARTICLES // 4
N_NCG = N_PARTICLES - N_CG

NW = 32
BLOCKS_PER_W = 4
NSLAB = BLOCKS_PER_W * N_DIM

IN_PLANE = N_BATCH * N_PARTICLES
SLAB = 16 * 2 * 8 * 128


def _sc_split(x_flat):
    mesh = plsc.VectorSubcoreMesh(core_axis_name="c", subcore_axis_name="s")

    @functools.partial(
        pl.kernel,
        mesh=mesh,
        compiler_params=pltpu.CompilerParams(needs_layout_passes=False),
        out_type=(
            jax.ShapeDtypeStruct((N_DIM * 8, 128, 8, 128), jnp.float32),
            jax.ShapeDtypeStruct((N_DIM * 24, 128, 8, 128), jnp.float32),
        ),
        scratch_types=[
            pltpu.VMEM((SLAB,), jnp.float32),
            pltpu.VMEM((SLAB,), jnp.float32),
            pltpu.VMEM((8, 8, 128), jnp.float32),
            pltpu.VMEM((8, 8, 128), jnp.float32),
            pltpu.VMEM((24, 8, 128), jnp.float32),
            pltpu.SemaphoreType.DMA,
            pltpu.SemaphoreType.DMA,
            pltpu.SemaphoreType.DMA,
            pltpu.SemaphoreType.DMA,
            pltpu.SemaphoreType.DMA,
        ],
    )
    def k(x_hbm, cg_hbm, ncg_hbm, vin0, vin1, vcg0, vcg1, vncg,
          sin0, sin1, sout0, sout1, sncg):
        wid = lax.axis_index("s") * 2 + lax.axis_index("c")
        vins = [vin0, vin1]
        vcgs = [vcg0, vcg1]
        sins = [sin0, sin1]
        souts = [sout0, sout1]

        i = lax.iota(jnp.int32, 16)
        pat0 = (i // 8) * 2048 + (i % 8) * 128
        pats = [pat0 + v * 4096 for v in range(8)]

        def slab_coords(s):
            c, p = divmod(s, 3)
            tbo = wid * BLOCKS_PER_W + c
            return p, tbo

        def start_in(s):
            p, tbo = slab_coords(s)
            src0 = p * IN_PLANE + tbo * SLAB
            return pltpu.async_copy(
                x_hbm.at[pl.ds(src0, SLAB)], vins[s % 2], sins[s % 2])

        in_descs = {0: start_in(0)}
        out_descs = {0: [], 1: []}
        ncg_desc = [None]
        for s in range(NSLAB):
            b = s % 2
            if s + 1 < NSLAB:
                in_descs[s + 1] = start_in(s + 1)
            in_descs[s].wait()
            for d in out_descs[b]:
                d.wait()
            if ncg_desc[0] is not None:
                ncg_desc[0].wait()
            vin, vcg = vins[b], vcgs[b]

            @plsc.parallel_loop(0, 64, unroll=4)
            def row_body(g, vin=vin, vcg=vcg, vncg=vncg):
                base = g * 4 + (g // 32) * 896
                for t in range(4):
                    qoff = base + t
                    if t == 0:
                        row, sub = g // 8, g % 8
                        dref = vcg
                    else:
                        j = 3 * g + t - 1
                        row, sub = j // 8, j % 8
                        dref = vncg
                    for v in range(8):
                        val = plsc.load_gather(vin, [pats[v] + qoff])
                        dref[row, sub, pl.ds(16 * v, 16)] = val

            p, tbo = slab_coords(s)
            out_descs[b] = [
                pltpu.async_copy(vcg, cg_hbm.at[pl.ds(p * 8, 8), tbo],
                                 souts[b]),
            ]
            ncg_desc[0] = pltpu.async_copy(
                vncg, ncg_hbm.at[pl.ds(p * 24, 24), tbo], sncg)
        for b in range(2):
            for d in out_descs[b]:
                d.wait()
        ncg_desc[0].wait()

    return k(x_flat)


@jax.jit
def kernel(input_coords):
    x5 = (input_coords.transpose(2, 0, 1)
          .reshape(N_DIM, 2048, 8, 2, 128)
          .transpose(0, 1, 3, 2, 4))
    x_flat = x5.reshape(N_DIM * IN_PLANE)
    cg_t, ncg_t = _sc_split(x_flat)
    cg = (cg_t.reshape(N_DIM, 8, 128, 8, 128)
          .transpose(0, 1, 3, 2, 4)
          .reshape(N_DIM, N_CG, N_BATCH)
          .transpose(2, 1, 0))
    ncg = (ncg_t.reshape(N_DIM, 24, 128, 8, 128)
           .transpose(0, 1, 3, 2, 4)
           .reshape(N_DIM, N_NCG, N_BATCH)
           .transpose(2, 1, 0))
    return (cg, ncg)


# speedup vs baseline: 102.2191x; 2.6034x over previous
import functools

import numpy as np
import jax
import jax.numpy as jnp
from jax import lax
from jax.experimental import pallas as pl
from jax.experimental.pallas import tpu as pltpu
from jax.experimental.pallas import tpu_sc as plsc

N_BATCH = 16384
N_PARTICLES = 256
N_DIM = 3
N_CG = N_PARTICLES // 4
N_NCG = N_PARTICLES - N_CG

NW = 32
BLOCKS_PER_W = 4
NSLAB = BLOCKS_PER_W * N_DIM
NHALF = NSLAB * 2

IN_PLANE = N_BATCH * N_PARTICLES
SLAB = 16 * 2 * 8 * 128
HSLAB_ROWS = 128
PITCH = 69


def _sc_split(x_flat):
    mesh = plsc.VectorSubcoreMesh(core_axis_name="c", subcore_axis_name="s")

    @functools.partial(
        pl.kernel,
        mesh=mesh,
        compiler_params=pltpu.CompilerParams(needs_layout_passes=False),
        out_type=(
            jax.ShapeDtypeStruct((N_DIM * 8, 128, 8, 128), jnp.float32),
            jax.ShapeDtypeStruct((N_DIM * 24, 128, 8, 128), jnp.float32),
        ),
        scratch_types=[
            pltpu.VMEM((HSLAB_ROWS, 128), jnp.float32),
            pltpu.VMEM((HSLAB_ROWS, 128), jnp.float32),
            pltpu.VMEM((N_PARTICLES * PITCH,), jnp.float32),
            pltpu.VMEM((8, 8, 128), jnp.float32),
            pltpu.VMEM((8, 8, 128), jnp.float32),
            pltpu.VMEM((24, 8, 128), jnp.float32),
            pltpu.SemaphoreType.DMA,
            pltpu.SemaphoreType.DMA,
            pltpu.SemaphoreType.DMA,
            pltpu.SemaphoreType.DMA,
            pltpu.SemaphoreType.DMA,
        ],
    )
    def k(x_hbm, cg_hbm, ncg_hbm, vin0, vin1, vtmp, vcg0, vcg1, vncg,
          sin0, sin1, sout0, sout1, sncg):
        wid = lax.axis_index("s") * 2 + lax.axis_index("c")
        vins = [vin0, vin1]
        vcgs = [vcg0, vcg1]
        sins = [sin0, sin1]
        souts = [sout0, sout1]

        ii = lax.iota(jnp.int32, 16)
        a, m = ii // 4, ii % 4
        idx69 = []
        for qv in range(16):
            q4 = 4 * qv + a
            row = jnp.where(m == 0, q4, 64 + 3 * q4 + m - 1)
            idx69.append(row * PITCH)

        def slab_coords(sl):
            c, p = divmod(sl, 3)
            tbo = wid * BLOCKS_PER_W + c
            return p, tbo

        def start_in(hs):
            sl, half = divmod(hs, 2)
            p, tbo = slab_coords(sl)
            row0 = pl.multiple_of(
                p * (IN_PLANE // 128) + tbo * (SLAB // 128) + half * 128, 128)
            return pltpu.async_copy(
                x_hbm.at[pl.ds(row0, HSLAB_ROWS)], vins[hs % 2], sins[hs % 2])

        in_descs = {0: start_in(0)}
        out_descs = {0: [], 1: []}
        ncg_desc = [None]
        for hs in range(NHALF):
            sl, half = divmod(hs, 2)
            b = sl % 2
            if hs + 1 < NHALF:
                in_descs[hs + 1] = start_in(hs + 1)
            in_descs[hs].wait()
            if half == 0:
                for d in out_descs[b]:
                    d.wait()
                out_descs[b] = []
                if ncg_desc[0] is not None:
                    ncg_desc[0].wait()
                    ncg_desc[0] = None
            vin, vcg = vins[hs % 2], vcgs[b]

            @plsc.parallel_loop(0, 64, unroll=2)
            def s1_body(bl, vin=vin):
                tb16 = (bl // 8) * 16 + bl % 8
                for qv in range(16):
                    rowb = tb16 + (qv // 8) * 8
                    val = vin[rowb, pl.ds((qv % 8) * 16, 16)]
                    plsc.store_scatter(vtmp, [idx69[qv] + bl], val)

            @plsc.parallel_loop(0, 64, unroll=4)
            def s2cg_body(r, vcg=vcg, half=half):
                for c in range(4):
                    vcg[r // 8, r % 8, pl.ds(half * 64 + 16 * c, 16)] = (
                        vtmp[pl.ds(r * PITCH + 16 * c, 16)])

            @plsc.parallel_loop(0, 192, unroll=4)
            def s2ncg_body(r, half=half):
                for c in range(4):
                    vncg[r // 8, r % 8, pl.ds(half * 64 + 16 * c, 16)] = (
                        vtmp[pl.ds((64 + r) * PITCH + 16 * c, 16)])

            if half == 1:
                p, tbo = slab_coords(sl)
                out_descs[b] = [
                    pltpu.async_copy(vcg, cg_hbm.at[pl.ds(p * 8, 8), tbo],
                                     souts[b]),
                ]
                ncg_desc[0] = pltpu.async_copy(
                    vncg, ncg_hbm.at[pl.ds(p * 24, 24), tbo], sncg)
        for b in range(2):
            for d in out_descs[b]:
                d.wait()
        if ncg_desc[0] is not None:
            ncg_desc[0].wait()

    return k(x_flat)


@jax.jit
def kernel(input_coords):
    x5 = (input_coords.transpose(2, 0, 1)
          .reshape(N_DIM, 2048, 8, 2, 128)
          .transpose(0, 1, 3, 2, 4))
    x_flat = x5.reshape(N_DIM * IN_PLANE // 128, 128)
    cg_t, ncg_t = _sc_split(x_flat)
    cg = (cg_t.reshape(N_DIM, 8, 128, 8, 128)
          .transpose(0, 1, 3, 2, 4)
          .reshape(N_DIM, N_CG, N_BATCH)
          .transpose(2, 1, 0))
    ncg = (ncg_t.reshape(N_DIM, 24, 128, 8, 128)
           .transpose(0, 1, 3, 2, 4)
           .reshape(N_DIM, N_NCG, N_BATCH)
           .transpose(2, 1, 0))
    return (cg, ncg)


# speedup vs baseline: 120.3273x; 1.1772x over previous
import functools

import numpy as np
import jax
import jax.numpy as jnp
from jax import lax
from jax.experimental import pallas as pl
from jax.experimental.pallas import tpu as pltpu
from jax.experimental.pallas import tpu_sc as plsc

N_BATCH = 16384
N_PARTICLES = 256
N_DIM = 3
N_CG = N_PARTICLES // 4
N_NCG = N_PARTICLES - N_CG

NW = 32
BLOCKS_PER_W = 4
NSLAB = BLOCKS_PER_W * N_DIM

IN_PLANE = N_BATCH * N_PARTICLES
SLAB = 16 * 2 * 8 * 128
HSLAB_ROWS = 128
PITCH = 69


def _sc_split(x_flat):
    mesh = plsc.VectorSubcoreMesh(core_axis_name="c", subcore_axis_name="s")

    @functools.partial(
        pl.kernel,
        mesh=mesh,
        compiler_params=pltpu.CompilerParams(needs_layout_passes=False),
        out_type=(
            jax.ShapeDtypeStruct((N_DIM * 8, 128, 8, 128), jnp.float32),
            jax.ShapeDtypeStruct((N_DIM * 24, 128, 8, 128), jnp.float32),
        ),
        scratch_types=[
            pltpu.VMEM((HSLAB_ROWS, 128), jnp.float32),
            pltpu.VMEM((HSLAB_ROWS, 128), jnp.float32),
            pltpu.VMEM((N_PARTICLES * PITCH,), jnp.float32),
            pltpu.VMEM((8, 8, 128), jnp.float32),
            pltpu.VMEM((24, 8, 128), jnp.float32),
            pltpu.SemaphoreType.DMA,
            pltpu.SemaphoreType.DMA,
            pltpu.SemaphoreType.DMA,
            pltpu.SemaphoreType.DMA,
        ],
    )
    def k(x_hbm, cg_hbm, ncg_hbm, vin0, vin1, vtmp, vcg, vncg,
          sin0, sin1, scg, sncg):
        wid = lax.axis_index("s") * 2 + lax.axis_index("c")
        vins = [vin0, vin1]
        sins = [sin0, sin1]

        ii = lax.iota(jnp.int32, 16)
        a, m = ii // 4, ii % 4
        idx69 = []
        for qv in range(16):
            q4 = 4 * qv + a
            row = jnp.where(m == 0, q4, 64 + 3 * q4 + m - 1)
            idx69.append(row * PITCH)

        def coords(sl):
            c = sl // 3
            p = sl - c * 3
            tbo = wid * BLOCKS_PER_W + c
            return p, tbo

        def start_in(sl, half):
            p, tbo = coords(sl)
            row0 = pl.multiple_of(
                p * (IN_PLANE // 128) + tbo * (SLAB // 128) + half * 128, 128)
            return pltpu.async_copy(
                x_hbm.at[pl.ds(row0, HSLAB_ROWS)], vins[half], sins[half])

        def wait_in(half):
            pltpu.make_async_copy(
                x_hbm.at[pl.ds(0, HSLAB_ROWS)], vins[half], sins[half]).wait()

        def wait_cg():
            pltpu.make_async_copy(
                vcg, cg_hbm.at[pl.ds(0, 8), 0], scg).wait()

        def wait_ncg():
            pltpu.make_async_copy(
                vncg, ncg_hbm.at[pl.ds(0, 24), 0], sncg).wait()

        def do_half(sl, half):
            vin = vins[half]

            @plsc.parallel_loop(0, 64, unroll=8)
            def s1_body(bl):
                tb16 = (bl // 8) * 16 + bl % 8
                for qv in range(16):
                    rowb = tb16 + (qv // 8) * 8
                    val = vin[rowb, pl.ds((qv % 8) * 16, 16)]
                    plsc.store_scatter(vtmp, [idx69[qv] + bl], val)

            @plsc.parallel_loop(0, 64, unroll=8)
            def s2cg_body(r):
                for c in range(4):
                    vcg[r // 8, r % 8, pl.ds(half * 64 + 16 * c, 16)] = (
                        vtmp[pl.ds(r * PITCH + 16 * c, 16)])

            @plsc.parallel_loop(0, 192, unroll=8)
            def s2ncg_body(r):
                for c in range(4):
                    vncg[r // 8, r % 8, pl.ds(half * 64 + 16 * c, 16)] = (
                        vtmp[pl.ds((64 + r) * PITCH + 16 * c, 16)])

        start_in(0, 0)

        def slab_body(sl, _):
            wait_in(0)
            start_in(sl, 1)

            @pl.when(sl > 0)
            def _drain_outs():
                wait_cg()
                wait_ncg()

            do_half(sl, 0)

            wait_in(1)
            nxt = jnp.minimum(sl + 1, NSLAB - 1)
            start_in(nxt, 0)
            do_half(sl, 1)

            p, tbo = coords(sl)
            pltpu.async_copy(vcg, cg_hbm.at[pl.ds(p * 8, 8), tbo], scg)
            pltpu.async_copy(vncg, ncg_hbm.at[pl.ds(p * 24, 24), tbo], sncg)
            return _

        lax.fori_loop(0, NSLAB, slab_body, None)
        wait_cg()
        wait_ncg()
        wait_in(0)

    return k(x_flat)


@jax.jit
def kernel(input_coords):
    x5 = (input_coords.transpose(2, 0, 1)
          .reshape(N_DIM, 2048, 8, 2, 128)
          .transpose(0, 1, 3, 2, 4))
    x_flat = x5.reshape(N_DIM * IN_PLANE // 128, 128)
    cg_t, ncg_t = _sc_split(x_flat)
    cg = (cg_t.reshape(N_DIM, 8, 128, 8, 128)
          .transpose(0, 1, 3, 2, 4)
          .reshape(N_DIM, N_CG, N_BATCH)
          .transpose(2, 1, 0))
    ncg = (ncg_t.reshape(N_DIM, 24, 128, 8, 128)
           .transpose(0, 1, 3, 2, 4)
           .reshape(N_DIM, N_NCG, N_BATCH)
           .transpose(2, 1, 0))
    return (cg, ncg)
